# baseline (speedup 1.0000x reference)
import functools

import jax
import jax.numpy as jnp
from jax import lax
from jax.experimental import pallas as pl
from jax.experimental.pallas import tpu as pltpu
from jax.experimental.pallas import tpu_sc as plsc

_BATCH = 16384
_RANK = 32
_NC = 2
_NS = 16
_NW = _NC * _NS
_BPW = _BATCH // _NW
_L = 16
_C = 32
_NCH = _BPW // _C


def _lfm_body(uid_hbm, iid_hbm, p_hbm, q_hbm, ub_hbm, ib_hbm, par_hbm,
              out_hbm, uidx_v, iidx_v, ut_v, it_v, du, di, ubv, ibv,
              par_v, out_v, sem_u, sem_i, sem_ub, sem_ib):
    wid = lax.axis_index("s") * _NC + lax.axis_index("c")
    base = wid * _BPW

    pltpu.sync_copy(uid_hbm.at[pl.ds(base, _BPW)], uidx_v)
    pltpu.sync_copy(iid_hbm.at[pl.ds(base, _BPW)], iidx_v)
    pltpu.sync_copy(par_hbm, par_v)

    cp_ub = pltpu.async_copy(ub_hbm.at[uidx_v], ubv, sem_ub)
    cp_ib = pltpu.async_copy(ib_hbm.at[iidx_v], ibv, sem_ib)

    def mk_tiles(g, _):
        u = uidx_v[pl.ds(g * _L, _L)]
        i = iidx_v[pl.ds(g * _L, _L)]
        ut_v[pl.ds(g * _L, _L)] = lax.shift_right_logical(u, 3)
        it_v[pl.ds(g * _L, _L)] = lax.shift_right_logical(i, 3)
        return _

    lax.fori_loop(0, _BPW // _L, mk_tiles, None)

    w0 = par_v[pl.ds(0, _L)]
    w1 = par_v[pl.ds(_L, _L)]
    tail = par_v[pl.ds(2 * _L, _L)]
    mu_b = tail[0] + tail[1]
    lane = lax.iota(jnp.int32, _L)

    def chunk(cc, _):
        cp_u = pltpu.async_copy(
            p_hbm.at[ut_v.at[pl.ds(cc * _C, _C)]], du, sem_u)
        cp_i = pltpu.async_copy(
            q_hbm.at[it_v.at[pl.ds(cc * _C, _C)]], di, sem_i)
        cp_u.wait()
        cp_i.wait()

        def grp(g, _):
            b = cc * _C + g * _L
            c16 = g * _L + lane
            su = jnp.bitwise_and(uidx_v[pl.ds(b, _L)], 7)
            si = jnp.bitwise_and(iidx_v[pl.ds(b, _L)], 7)
            acc = jnp.zeros((_L,), jnp.float32)
            for j in range(_RANK):
                j16 = jnp.full((_L,), j, jnp.int32)
                up = plsc.load_gather(du, [c16, su, j16])
                it = plsc.load_gather(di, [c16, si, j16])
                x = up * it
                x = jnp.maximum(x, x * 0.3)
                wj = w0[j] if j < _L else w1[j - _L]
                acc = acc + x * wj
            out_v[pl.ds(b, _L)] = acc
            return _

        lax.fori_loop(0, _C // _L, grp, None)
        return _

    lax.fori_loop(0, _NCH, chunk, None)

    cp_ub.wait()
    cp_ib.wait()

    def finish(g, _):
        sl = pl.ds(g * _L, _L)
        r = out_v[sl] + ubv[sl] + ibv[sl] + mu_b
        out_v[sl] = jnp.clip(r, 1.0, 5.0)
        return _

    lax.fori_loop(0, _BPW // _L, finish, None)

    pltpu.sync_copy(out_v, out_hbm.at[pl.ds(base, _BPW)])


@jax.jit
def _lfm(user_ids, item_ids, P3, Q3, user_bias, item_bias, params):
    mesh = plsc.VectorSubcoreMesh(core_axis_name="c", subcore_axis_name="s")
    return pl.kernel(
        _lfm_body,
        out_type=jax.ShapeDtypeStruct((_BATCH,), jnp.float32),
        mesh=mesh,
        compiler_params=pltpu.CompilerParams(
            needs_layout_passes=False, use_tc_tiling_on_sc=False
        ),
        scratch_types=[
            pltpu.VMEM((_BPW,), jnp.int32),
            pltpu.VMEM((_BPW,), jnp.int32),
            pltpu.VMEM((_BPW,), jnp.int32),
            pltpu.VMEM((_BPW,), jnp.int32),
            pltpu.VMEM((_C, 8, _RANK), jnp.float32),
            pltpu.VMEM((_C, 8, _RANK), jnp.float32),
            pltpu.VMEM((_BPW,), jnp.float32),
            pltpu.VMEM((_BPW,), jnp.float32),
            pltpu.VMEM((3 * _L,), jnp.float32),
            pltpu.VMEM((_BPW,), jnp.float32),
            pltpu.SemaphoreType.DMA,
            pltpu.SemaphoreType.DMA,
            pltpu.SemaphoreType.DMA,
            pltpu.SemaphoreType.DMA,
        ],
    )(user_ids, item_ids, P3, Q3, user_bias, item_bias, params)


def kernel(user_ids, item_ids, P, Q, mu, user_bias, item_bias, W, b):
    params = jnp.concatenate(
        [W.reshape(-1), mu, b, jnp.zeros((3 * _L - _RANK - 2,), jnp.float32)]
    )
    P3 = P.reshape(P.shape[0] // 8, 8, _RANK)
    Q3 = Q.reshape(Q.shape[0] // 8, 8, _RANK)
    return _lfm(user_ids.astype(jnp.int32), item_ids.astype(jnp.int32),
                P3, Q3, user_bias, item_bias, params)


# speedup vs baseline: 2.5769x; 2.5769x over previous
import functools

import jax
import jax.numpy as jnp
from jax import lax
from jax.experimental import pallas as pl
from jax.experimental.pallas import tpu as pltpu
from jax.experimental.pallas import tpu_sc as plsc

_BATCH = 16384
_RANK = 32
_NC = 2
_NS = 16
_NW = _NC * _NS
_BPW = _BATCH // _NW
_L = 16
_NG = _BPW // _L
_NSEM = 4


def _lfm_body(uid_hbm, iid_hbm, p_hbm, q_hbm, ub_hbm, ib_hbm, par_hbm,
              out_hbm, uidx_v, iidx_v, ring_u, ring_i, ubv, ibv, par_v,
              out_v, sems_u, sems_i, sem_ub, sem_ib):
    wid = lax.axis_index("s") * _NC + lax.axis_index("c")
    base = wid * _BPW

    pltpu.sync_copy(uid_hbm.at[pl.ds(base, _BPW)], uidx_v)
    pltpu.sync_copy(iid_hbm.at[pl.ds(base, _BPW)], iidx_v)
    pltpu.sync_copy(par_hbm, par_v)

    cp_ub = pltpu.async_copy(ub_hbm.at[uidx_v], ubv, sem_ub)
    cp_ib = pltpu.async_copy(ib_hbm.at[iidx_v], ibv, sem_ib)

    w0 = par_v[pl.ds(0, _L)]
    w1 = par_v[pl.ds(_L, _L)]
    tail = par_v[pl.ds(2 * _L, _L)]
    mu_b = tail[0] + tail[1]
    lane = lax.iota(jnp.int32, _L)

    def issue(g, buf):
        u16 = uidx_v[pl.ds(g * _L, _L)]
        i16 = iidx_v[pl.ds(g * _L, _L)]
        tu16 = lax.shift_right_logical(u16, 3)
        ti16 = lax.shift_right_logical(i16, 3)
        su16 = jnp.bitwise_and(u16, 7)
        si16 = jnp.bitwise_and(i16, 7)
        for k in range(_L):
            pltpu.async_copy(p_hbm.at[tu16[k], su16[k]], ring_u.at[buf, k],
                             sems_u.at[k % _NSEM])
            pltpu.async_copy(q_hbm.at[ti16[k], si16[k]], ring_i.at[buf, k],
                             sems_i.at[k % _NSEM])

    def drain(buf):
        for k in range(_L):
            pltpu.make_async_copy(p_hbm.at[0, 0], ring_u.at[buf, k],
                                  sems_u.at[k % _NSEM]).wait()
            pltpu.make_async_copy(q_hbm.at[0, 0], ring_i.at[buf, k],
                                  sems_i.at[k % _NSEM]).wait()

    issue(0, 0)

    def grp(g, _):
        buf = jnp.bitwise_and(g, 1)

        @pl.when(g + 1 < _NG)
        def _():
            issue(g + 1, 1 - buf)

        drain(buf)
        acc = jnp.zeros((_L,), jnp.float32)
        b16 = jnp.full((_L,), buf, jnp.int32)
        for j in range(_RANK):
            j16 = jnp.full((_L,), j, jnp.int32)
            up = plsc.load_gather(ring_u, [b16, lane, j16])
            it = plsc.load_gather(ring_i, [b16, lane, j16])
            x = up * it
            x = jnp.maximum(x, x * 0.3)
            wj = w0[j] if j < _L else w1[j - _L]
            acc = acc + x * wj
        out_v[pl.ds(g * _L, _L)] = acc
        return _

    lax.fori_loop(0, _NG, grp, None)

    cp_ub.wait()
    cp_ib.wait()

    def finish(g, _):
        sl = pl.ds(g * _L, _L)
        r = out_v[sl] + ubv[sl] + ibv[sl] + mu_b
        out_v[sl] = jnp.clip(r, 1.0, 5.0)
        return _

    lax.fori_loop(0, _NG, finish, None)

    pltpu.sync_copy(out_v, out_hbm.at[pl.ds(base, _BPW)])


@jax.jit
def _lfm(user_ids, item_ids, P3, Q3, user_bias, item_bias, params):
    mesh = plsc.VectorSubcoreMesh(core_axis_name="c", subcore_axis_name="s")
    return pl.kernel(
        _lfm_body,
        out_type=jax.ShapeDtypeStruct((_BATCH,), jnp.float32),
        mesh=mesh,
        compiler_params=pltpu.CompilerParams(needs_layout_passes=False),
        scratch_types=[
            pltpu.VMEM((_BPW,), jnp.int32),
            pltpu.VMEM((_BPW,), jnp.int32),
            pltpu.VMEM((2, _L, _RANK), jnp.float32),
            pltpu.VMEM((2, _L, _RANK), jnp.float32),
            pltpu.VMEM((_BPW,), jnp.float32),
            pltpu.VMEM((_BPW,), jnp.float32),
            pltpu.VMEM((3 * _L,), jnp.float32),
            pltpu.VMEM((_BPW,), jnp.float32),
            pltpu.SemaphoreType.DMA((_NSEM,)),
            pltpu.SemaphoreType.DMA((_NSEM,)),
            pltpu.SemaphoreType.DMA,
            pltpu.SemaphoreType.DMA,
        ],
    )(user_ids, item_ids, P3, Q3, user_bias, item_bias, params)


def kernel(user_ids, item_ids, P, Q, mu, user_bias, item_bias, W, b):
    params = jnp.concatenate(
        [W.reshape(-1), mu, b, jnp.zeros((3 * _L - _RANK - 2,), jnp.float32)]
    )
    P3 = P.reshape(P.shape[0] // 8, 8, _RANK)
    Q3 = Q.reshape(Q.shape[0] // 8, 8, _RANK)
    return _lfm(user_ids.astype(jnp.int32), item_ids.astype(jnp.int32),
                P3, Q3, user_bias, item_bias, params)
